# Initial kernel scaffold; baseline (speedup 1.0000x reference)
import jax
import jax.numpy as jnp
from jax.experimental import pallas as pl


def kernel(inputs, item_rating_logits):
    raise NotImplementedError("write your pallas kernel here")


# speedup vs baseline: 172.7523x; 172.7523x over previous
import functools

import jax
import jax.numpy as jnp
from jax import lax
from jax.experimental import pallas as pl
from jax.experimental.pallas import tpu as pltpu
from jax.experimental.pallas import tpu_sc as plsc

NUM_ITEMS = 1_000_000
TBL = 1 << 20
B, S = 16384, 200
N_IDX = B * S
NC, NS = 2, 16
NW = NC * NS
PER_W = N_IDX // NW
WIN = 12800
NWIN = PER_W // WIN


def _sigmoid_body(x_ref, o_ref):
    o_ref[...] = jax.nn.sigmoid(8.0 * x_ref[...])


def _build_table(item_rating_logits):
    pad_lo = jnp.full((1,), -1e30, jnp.float32)
    pad_hi = jnp.full((TBL - NUM_ITEMS,), -1e30, jnp.float32)
    x = jnp.concatenate([pad_lo, item_rating_logits, pad_hi]).reshape(1024, 1024)
    table = pl.pallas_call(
        _sigmoid_body,
        out_shape=jax.ShapeDtypeStruct((1024, 1024), jnp.float32),
    )(x)
    return table.reshape(TBL)


@functools.partial(
    pl.kernel,
    out_type=jax.ShapeDtypeStruct((N_IDX,), jnp.float32),
    mesh=plsc.VectorSubcoreMesh(core_axis_name="c", subcore_axis_name="s"),
    scratch_types=[
        pltpu.VMEM_SHARED((TBL,), jnp.float32),
        pltpu.VMEM((WIN,), jnp.int32),
        pltpu.VMEM((WIN,), jnp.float32),
    ],
)
def _gather(table_hbm, idx_hbm, out_hbm, tbl_sp, idx_v, out_v):
    c = lax.axis_index("c")
    s = lax.axis_index("s")
    seg = TBL // NS
    pltpu.sync_copy(table_hbm.at[pl.ds(s * seg, seg)], tbl_sp.at[pl.ds(s * seg, seg)])
    plsc.subcore_barrier()

    wid = s * NC + c
    base = wid * PER_W

    def body(w, carry):
        off = base + w * WIN
        pltpu.sync_copy(idx_hbm.at[pl.ds(off, WIN)], idx_v)
        pltpu.sync_copy(tbl_sp.at[idx_v], out_v)
        pltpu.sync_copy(out_v, out_hbm.at[pl.ds(off, WIN)])
        return carry

    lax.fori_loop(0, NWIN, body, 0)


def kernel(inputs, item_rating_logits):
    table = _build_table(item_rating_logits)
    idx = inputs.reshape(N_IDX)
    out = _gather(table, idx)
    return out.reshape(B, S)


# speedup vs baseline: 210.1563x; 1.2165x over previous
import functools

import jax
import jax.numpy as jnp
from jax import lax
from jax.experimental import pallas as pl
from jax.experimental.pallas import tpu as pltpu
from jax.experimental.pallas import tpu_sc as plsc

NUM_ITEMS = 1_000_000
TBL = 1 << 20
TR, TC_ = TBL // 128, 128
ROWS, COLS = 16384, 200
NC, NS = 2, 16
NW = NC * NS
ROWS_PER_W = ROWS // NW
RW = 64
NWIN = ROWS_PER_W // RW


def _table_body(x_ref, o_ref):
    x = x_ref[...]
    prev_rows = jnp.concatenate(
        [jnp.full((1, TC_), -1e30, jnp.float32), x[:-1, :]], axis=0
    )
    col = lax.broadcasted_iota(jnp.int32, (TR, TC_), 1)
    row = lax.broadcasted_iota(jnp.int32, (TR, TC_), 0)
    xsel = jnp.where(col == TC_ - 1, prev_rows, x)
    shifted = pltpu.roll(xsel, 1, axis=1)
    tbl = jax.nn.sigmoid(8.0 * shifted)
    o_ref[...] = jnp.where((row == 0) & (col == 0), 0.0, tbl)


N_IDX = ROWS * COLS
WIN = RW * COLS
@functools.partial(
    pl.kernel,
    out_type=jax.ShapeDtypeStruct((N_IDX,), jnp.float32),
    mesh=plsc.VectorSubcoreMesh(core_axis_name="c", subcore_axis_name="s"),
    scratch_types=[
        pltpu.VMEM_SHARED((TBL,), jnp.float32),
        pltpu.VMEM((WIN,), jnp.int32),
        pltpu.VMEM((WIN,), jnp.float32),
    ],
)
def _gather(table_hbm, idx_hbm, out_hbm, tbl_sp, idx_v, out_v):
    c = lax.axis_index("c")
    s = lax.axis_index("s")
    seg = TBL // NS
    pltpu.sync_copy(table_hbm.at[pl.ds(s * seg, seg)], tbl_sp.at[pl.ds(s * seg, seg)])
    plsc.subcore_barrier()

    wid = s * NC + c
    base = wid * (N_IDX // NW)

    def body(w, carry):
        off = base + w * WIN
        pltpu.sync_copy(idx_hbm.at[pl.ds(off, WIN)], idx_v)
        pltpu.sync_copy(tbl_sp.at[idx_v], out_v)
        pltpu.sync_copy(out_v, out_hbm.at[pl.ds(off, WIN)])
        return carry

    lax.fori_loop(0, NWIN, body, 0)


def kernel(inputs, item_rating_logits):
    pad = jnp.full((TBL - NUM_ITEMS + 1,), -1e30, jnp.float32)
    x = jnp.concatenate([item_rating_logits, pad]).reshape(TR, TC_)
    table = pl.pallas_call(
        _table_body,
        out_shape=jax.ShapeDtypeStruct((TR, TC_), jnp.float32),
    )(x)
    out = _gather(table.reshape(TBL), inputs.reshape(N_IDX))
    return out.reshape(ROWS, COLS)
